# Initial kernel scaffold; baseline (speedup 1.0000x reference)
import functools

import jax
import jax.numpy as jnp
from jax import lax
from jax.experimental import pallas as pl
from jax.experimental.pallas import tpu as pltpu
from jax.experimental.pallas import tpu_sc as plsc

NC = 2
NS = 16
LANES = 16
CH = 80


def _sc_aggregate(x, src2, dst2, w2):
    N, D = x.shape
    n_chunks, ch = src2.shape
    nw = NC * NS
    rows_per_tile = n_chunks // nw
    n_out_blocks = N // ch
    blocks_per_tile = (n_out_blocks + NS - 1) // NS
    mesh = plsc.VectorSubcoreMesh(core_axis_name="c", subcore_axis_name="s")

    @functools.partial(
        pl.kernel,
        out_type=jax.ShapeDtypeStruct((NC * N, D), jnp.float32),
        mesh=mesh,
        scratch_types=[
            pltpu.VMEM((rows_per_tile, ch), jnp.int32),
            pltpu.VMEM((rows_per_tile, ch), jnp.int32),
            pltpu.VMEM((rows_per_tile, ch), jnp.float32),
            pltpu.VMEM((ch, D), jnp.float32),
            pltpu.VMEM_SHARED((N, D), jnp.float32),
            pltpu.SemaphoreType.DMA,
        ],
    )
    def k(x_hbm, src_hbm, dst_hbm, w_hbm, out_hbm, src_v, dst_v, w_v,
          rows_v, acc_sh, sem):
        cid = lax.axis_index("c")
        sid = lax.axis_index("s")
        wid = sid * NC + cid
        base = wid * rows_per_tile

        pltpu.sync_copy(src_hbm.at[pl.ds(base, rows_per_tile)], src_v)
        pltpu.sync_copy(dst_hbm.at[pl.ds(base, rows_per_tile)], dst_v)
        pltpu.sync_copy(w_hbm.at[pl.ds(base, rows_per_tile)], w_v)

        @pl.loop(0, ch)
        def _(e):
            for kk in range(D // LANES):
                rows_v[e, pl.ds(kk * LANES, LANES)] = jnp.zeros(
                    (LANES,), jnp.float32)

        @pl.loop(0, blocks_per_tile)
        def _(i):
            blk = sid + NS * i

            @pl.when(blk < n_out_blocks)
            def _():
                pltpu.sync_copy(rows_v, acc_sh.at[pl.ds(blk * ch, ch)])

        plsc.subcore_barrier()

        @pl.loop(0, rows_per_tile)
        def _(j):
            pltpu.async_copy(x_hbm.at[src_v.at[j]], rows_v, sem).wait()

            @pl.loop(0, ch)
            def _(e):
                wb = lax.broadcast(w_v[j, e], (LANES,))
                for kk in range(D // LANES):
                    sl = (e, pl.ds(kk * LANES, LANES))
                    rows_v[sl] = rows_v[sl] * wb

            pltpu.sync_copy(rows_v, acc_sh.at[dst_v.at[j]], add=True)

        plsc.subcore_barrier()

        @pl.loop(0, blocks_per_tile)
        def _(i):
            blk = sid + NS * i

            @pl.when(blk < n_out_blocks)
            def _():
                pltpu.sync_copy(
                    acc_sh.at[pl.ds(blk * ch, ch)],
                    out_hbm.at[pl.ds(cid * N + blk * ch, ch)])

    return k(x, src2, dst2, w2)


def _tc_combine_matmul(agg, W, b):
    two_n, d_in = agg.shape
    n = two_n // 2
    d_out = W.shape[1]
    blk = 1000
    grid = n // blk

    def body(p0_ref, p1_ref, w_ref, b_ref, o_ref):
        s = p0_ref[...] + p1_ref[...]
        o_ref[...] = (
            jnp.dot(s, w_ref[...], preferred_element_type=jnp.float32)
            + b_ref[...]
        )

    return pl.pallas_call(
        body,
        grid=(grid,),
        in_specs=[
            pl.BlockSpec((blk, d_in), lambda i: (i, 0)),
            pl.BlockSpec((blk, d_in), lambda i: (i, 0)),
            pl.BlockSpec((d_in, d_out), lambda i: (0, 0)),
            pl.BlockSpec((1, d_out), lambda i: (0, 0)),
        ],
        out_specs=pl.BlockSpec((blk, d_out), lambda i: (i, 0)),
        out_shape=jax.ShapeDtypeStruct((n, d_out), jnp.float32),
    )(agg[:n], agg[n:], W, b.reshape(1, d_out))


@jax.jit
def kernel(x, edge_index, edge_weight, W, b):
    e = edge_weight.shape[0]
    n_chunks = e // CH
    src2 = edge_index[0].reshape(n_chunks, CH)
    dst2 = edge_index[1].reshape(n_chunks, CH)
    w2 = edge_weight.reshape(n_chunks, CH)
    agg = _sc_aggregate(x, src2, dst2, w2)
    return _tc_combine_matmul(agg, W, b)


# speedup vs baseline: 2.8203x; 2.8203x over previous
import functools

import jax
import jax.numpy as jnp
from jax import lax
from jax.experimental import pallas as pl
from jax.experimental.pallas import tpu as pltpu
from jax.experimental.pallas import tpu_sc as plsc

NC = 2
NS = 16
LANES = 16
CH = 80


def _sc_aggregate(x0, x1, src3, dst3, w3):
    N, Dh = x0.shape
    ns, rows_per_tile, ch = src3.shape
    n_out_blocks = N // ch
    blocks_per_tile = (n_out_blocks + NS - 1) // NS
    mesh = plsc.VectorSubcoreMesh(core_axis_name="c", subcore_axis_name="s")

    @functools.partial(
        pl.kernel,
        out_type=jax.ShapeDtypeStruct((NC * N, Dh), jnp.float32),
        mesh=mesh,
        compiler_params=pltpu.CompilerParams(use_tc_tiling_on_sc=False),
        scratch_types=[
            pltpu.VMEM((rows_per_tile, ch), jnp.int32),
            pltpu.VMEM((rows_per_tile, ch), jnp.int32),
            pltpu.VMEM((rows_per_tile, ch), jnp.float32),
            pltpu.VMEM((ch, Dh), jnp.float32),
            pltpu.VMEM_SHARED((N, Dh), jnp.float32),
            pltpu.SemaphoreType.DMA,
        ],
    )
    def k(x0_hbm, x1_hbm, src_hbm, dst_hbm, w_hbm, out_hbm, src_v, dst_v, w_v,
          rows_v, acc_sh, sem):
        cid = lax.axis_index("c")
        sid = lax.axis_index("s")

        pltpu.sync_copy(src_hbm.at[sid], src_v)
        pltpu.sync_copy(dst_hbm.at[sid], dst_v)
        pltpu.sync_copy(w_hbm.at[sid], w_v)

        @pl.loop(0, ch)
        def _(e):
            for kk in range(Dh // LANES):
                rows_v[e, pl.ds(kk * LANES, LANES)] = jnp.zeros(
                    (LANES,), jnp.float32)

        @pl.loop(0, blocks_per_tile)
        def _(i):
            blk = sid + NS * i

            @pl.when(blk < n_out_blocks)
            def _():
                pltpu.sync_copy(rows_v, acc_sh.at[pl.ds(blk * ch, ch)])

        plsc.subcore_barrier()

        @pl.loop(0, rows_per_tile)
        def _(j):
            @pl.when(cid == 0)
            def _():
                pltpu.async_copy(x0_hbm.at[src_v.at[j]], rows_v, sem).wait()

            @pl.when(cid == 1)
            def _():
                pltpu.async_copy(x1_hbm.at[src_v.at[j]], rows_v, sem).wait()

            @pl.loop(0, ch, step=LANES)
            def _(e0):
                wvec = w_v[j, pl.ds(e0, LANES)]
                for i in range(LANES):
                    wb = lax.broadcast(wvec[i], (LANES,))
                    for kk in range(Dh // LANES):
                        sl = (e0 + i, pl.ds(kk * LANES, LANES))
                        rows_v[sl] = rows_v[sl] * wb

            pltpu.sync_copy(rows_v, acc_sh.at[dst_v.at[j]], add=True)

        plsc.subcore_barrier()

        @pl.loop(0, blocks_per_tile)
        def _(i):
            blk = sid + NS * i

            @pl.when(blk < n_out_blocks)
            def _():
                pltpu.sync_copy(
                    acc_sh.at[pl.ds(blk * ch, ch)],
                    out_hbm.at[pl.ds(cid * N + blk * ch, ch)])

    return k(x0, x1, src3, dst3, w3)


def _tc_combine_matmul(agg, W, b):
    two_n, dh = agg.shape
    n = two_n // 2
    d_out = W.shape[1]
    blk = 1000
    grid = n // blk

    def body(a0_ref, a1_ref, w0_ref, w1_ref, b_ref, o_ref):
        o_ref[...] = (
            jnp.dot(a0_ref[...], w0_ref[...],
                    preferred_element_type=jnp.float32)
            + jnp.dot(a1_ref[...], w1_ref[...],
                      preferred_element_type=jnp.float32)
            + b_ref[...]
        )

    return pl.pallas_call(
        body,
        grid=(grid,),
        in_specs=[
            pl.BlockSpec((blk, dh), lambda i: (i, 0)),
            pl.BlockSpec((blk, dh), lambda i: (i, 0)),
            pl.BlockSpec((dh, d_out), lambda i: (0, 0)),
            pl.BlockSpec((dh, d_out), lambda i: (0, 0)),
            pl.BlockSpec((1, d_out), lambda i: (0, 0)),
        ],
        out_specs=pl.BlockSpec((blk, d_out), lambda i: (i, 0)),
        out_shape=jax.ShapeDtypeStruct((n, d_out), jnp.float32),
    )(agg[:n], agg[n:], W[:dh], W[dh:], b.reshape(1, d_out))


@jax.jit
def kernel(x, edge_index, edge_weight, W, b):
    e = edge_weight.shape[0]
    d = x.shape[1]
    dh = d // 2
    rows_per_tile = e // (NS * CH)
    shape3 = (NS, rows_per_tile, CH)
    src3 = edge_index[0].reshape(shape3)
    dst3 = edge_index[1].reshape(shape3)
    w3 = edge_weight.reshape(shape3)
    x0 = x[:, :dh]
    x1 = x[:, dh:]
    agg = _sc_aggregate(x0, x1, src3, dst3, w3)
    return _tc_combine_matmul(agg, W, b)


# speedup vs baseline: 4.0145x; 1.4234x over previous
import functools

import jax
import jax.numpy as jnp
from jax import lax
from jax.experimental import pallas as pl
from jax.experimental.pallas import tpu as pltpu
from jax.experimental.pallas import tpu_sc as plsc

NC = 2
NS = 16
LANES = 16
CH = 80


def _sc_aggregate(x0, x1, src3, dst3, w3):
    N, Dh = x0.shape
    ns, rows_per_tile, ch = src3.shape
    n_out_blocks = N // ch
    blocks_per_tile = (n_out_blocks + NS - 1) // NS
    mesh = plsc.VectorSubcoreMesh(core_axis_name="c", subcore_axis_name="s")

    @functools.partial(
        pl.kernel,
        out_type=jax.ShapeDtypeStruct((NC * N, Dh), jnp.float32),
        mesh=mesh,
        compiler_params=pltpu.CompilerParams(use_tc_tiling_on_sc=False),
        scratch_types=[
            pltpu.VMEM((rows_per_tile, ch), jnp.int32),
            pltpu.VMEM((rows_per_tile, ch), jnp.int32),
            pltpu.VMEM((rows_per_tile, ch), jnp.float32),
            pltpu.VMEM((ch, Dh), jnp.float32),
            pltpu.VMEM((ch, Dh), jnp.float32),
            pltpu.VMEM_SHARED((N, Dh), jnp.float32),
            pltpu.SemaphoreType.DMA,
            pltpu.SemaphoreType.DMA,
        ],
    )
    def k(x0_hbm, x1_hbm, src_hbm, dst_hbm, w_hbm, out_hbm, src_v, dst_v, w_v,
          rows_a, rows_b, acc_sh, sem_a, sem_b):
        cid = lax.axis_index("c")
        sid = lax.axis_index("s")

        pltpu.sync_copy(src_hbm.at[sid], src_v)
        pltpu.sync_copy(dst_hbm.at[sid], dst_v)
        pltpu.sync_copy(w_hbm.at[sid], w_v)

        @pl.loop(0, ch)
        def _(e):
            for kk in range(Dh // LANES):
                rows_a[e, pl.ds(kk * LANES, LANES)] = jnp.zeros(
                    (LANES,), jnp.float32)

        @pl.loop(0, blocks_per_tile)
        def _(i):
            blk = sid + NS * i

            @pl.when(blk < n_out_blocks)
            def _():
                pltpu.sync_copy(rows_a, acc_sh.at[pl.ds(blk * ch, ch)])

        def gather_start(j, buf, sem):
            @pl.when(cid == 0)
            def _():
                pltpu.async_copy(x0_hbm.at[src_v.at[j]], buf, sem)

            @pl.when(cid == 1)
            def _():
                pltpu.async_copy(x1_hbm.at[src_v.at[j]], buf, sem)

        def gather_wait(j, buf, sem):
            pltpu.make_async_copy(x0_hbm.at[src_v.at[j]], buf, sem).wait()

        def scale(j, buf):
            @pl.loop(0, ch, step=LANES)
            def _(e0):
                wvec = w_v[j, pl.ds(e0, LANES)]
                for i in range(LANES):
                    wb = lax.broadcast(wvec[i], (LANES,))
                    for kk in range(Dh // LANES):
                        sl = (e0 + i, pl.ds(kk * LANES, LANES))
                        buf[sl] = buf[sl] * wb

        gather_start(0, rows_a, sem_a)
        plsc.subcore_barrier()

        @pl.loop(0, rows_per_tile, step=2)
        def _(j):
            gather_wait(j, rows_a, sem_a)
            gather_start(j + 1, rows_b, sem_b)
            scale(j, rows_a)
            pltpu.sync_copy(rows_a, acc_sh.at[dst_v.at[j]], add=True)

            gather_wait(j + 1, rows_b, sem_b)

            @pl.when(j + 2 < rows_per_tile)
            def _():
                gather_start(j + 2, rows_a, sem_a)

            scale(j + 1, rows_b)
            pltpu.sync_copy(rows_b, acc_sh.at[dst_v.at[j + 1]], add=True)

        plsc.subcore_barrier()

        @pl.loop(0, blocks_per_tile)
        def _(i):
            blk = sid + NS * i

            @pl.when(blk < n_out_blocks)
            def _():
                pltpu.sync_copy(
                    acc_sh.at[pl.ds(blk * ch, ch)],
                    out_hbm.at[pl.ds(cid * N + blk * ch, ch)])

    return k(x0, x1, src3, dst3, w3)


def _tc_combine_matmul(agg, W, b):
    two_n, dh = agg.shape
    n = two_n // 2
    d_out = W.shape[1]
    blk = 1000
    grid = n // blk

    def body(a0_ref, a1_ref, w0_ref, w1_ref, b_ref, o_ref):
        o_ref[...] = (
            jnp.dot(a0_ref[...], w0_ref[...],
                    preferred_element_type=jnp.float32)
            + jnp.dot(a1_ref[...], w1_ref[...],
                      preferred_element_type=jnp.float32)
            + b_ref[...]
        )

    return pl.pallas_call(
        body,
        grid=(grid,),
        in_specs=[
            pl.BlockSpec((blk, dh), lambda i: (i, 0)),
            pl.BlockSpec((blk, dh), lambda i: (i, 0)),
            pl.BlockSpec((dh, d_out), lambda i: (0, 0)),
            pl.BlockSpec((dh, d_out), lambda i: (0, 0)),
            pl.BlockSpec((1, d_out), lambda i: (0, 0)),
        ],
        out_specs=pl.BlockSpec((blk, d_out), lambda i: (i, 0)),
        out_shape=jax.ShapeDtypeStruct((n, d_out), jnp.float32),
    )(agg[:n], agg[n:], W[:dh], W[dh:], b.reshape(1, d_out))


@jax.jit
def kernel(x, edge_index, edge_weight, W, b):
    e = edge_weight.shape[0]
    d = x.shape[1]
    dh = d // 2
    rows_per_tile = e // (NS * CH)
    shape3 = (NS, rows_per_tile, CH)
    src3 = edge_index[0].reshape(shape3)
    dst3 = edge_index[1].reshape(shape3)
    w3 = edge_weight.reshape(shape3)
    x0 = x[:, :dh]
    x1 = x[:, dh:]
    agg = _sc_aggregate(x0, x1, src3, dst3, w3)
    return _tc_combine_matmul(agg, W, b)
